# baseline (speedup 1.0000x reference)
import functools

import jax
import jax.numpy as jnp
from jax import lax
from jax.experimental import pallas as pl
from jax.experimental.pallas import tpu as pltpu
from jax.experimental.pallas import tpu_sc as plsc

NC, NS, LANES = 2, 16, 16
NW = NC * NS
D = 128
G_OUT = 128
CHUNK = 128
SUPER = 16

_SC_PARAMS = pltpu.CompilerParams(needs_layout_passes=False)


def _round_up(a, b):
    return (a + b - 1) // b * b


def _edge_scatter_loop(table, acc, srcm, dstm, srcbuf, dstbuf, rows0, rows1,
                       sem0, sem1, super0, nsupers):
    rows = (rows0, rows1)
    sems = (sem0, sem1)

    @pl.loop(0, nsupers)
    def _(jo):
        row0 = (super0 + jo) * SUPER
        pltpu.sync_copy(srcm.at[pl.ds(row0, SUPER)], srcbuf)
        pltpu.sync_copy(dstm.at[pl.ds(row0, SUPER)], dstbuf)
        pend = pltpu.async_copy(table.at[srcbuf.at[0]], rows[0], sems[0])
        for jj in range(SUPER):
            pend.wait()
            if jj + 1 < SUPER:
                pend = pltpu.async_copy(table.at[srcbuf.at[jj + 1]],
                                        rows[(jj + 1) % 2],
                                        sems[(jj + 1) % 2])
            pltpu.sync_copy(rows[jj % 2], acc.at[dstbuf.at[jj]], add=True)


def _sc_deg(dstm, z1d, np_, ep):
    chunks = ep // CHUNK
    nsupers = chunks // (NW * SUPER)
    mesh = plsc.VectorSubcoreMesh(core_axis_name="c", subcore_axis_name="s",
                                  num_cores=NC, num_subcores=NS)

    @functools.partial(
        pl.kernel,
        out_type=jax.ShapeDtypeStruct((NW, np_), jnp.float32),
        mesh=mesh,
        scratch_types=[
            pltpu.VMEM((SUPER, CHUNK), jnp.int32),
            pltpu.VMEM((np_,), jnp.float32),
        ],
        compiler_params=_SC_PARAMS)
    def kern(dstm_hbm, z1d_hbm, out_deg, dstbuf, degv):
        c = lax.axis_index("c")
        s = lax.axis_index("s")
        w = c * NS + s
        pltpu.sync_copy(z1d_hbm, degv)

        @pl.loop(0, nsupers)
        def _(jo):
            row0 = (w * nsupers + jo) * SUPER
            pltpu.sync_copy(dstm_hbm.at[pl.ds(row0, SUPER)], dstbuf)
            for jj in range(SUPER):
                for q in range(CHUNK // LANES):
                    idx = dstbuf[jj, pl.ds(q * LANES, LANES)]
                    plsc.addupdate_scatter(
                        degv, [idx], jnp.ones((LANES,), jnp.float32))

        pltpu.sync_copy(degv, out_deg.at[w])

    return kern(dstm, z1d)


def _sc_scatter_two_tables(t0, t1, srcm, dstm, zrow, np_, ep):
    rpt = np_ // NS
    chunks = ep // CHUNK
    nsupers = chunks // (NS * SUPER)

    mesh = plsc.VectorSubcoreMesh(core_axis_name="c", subcore_axis_name="s",
                                  num_cores=NC, num_subcores=NS)

    @functools.partial(
        pl.kernel,
        out_type=jax.ShapeDtypeStruct((NC, np_, D), jnp.float32),
        mesh=mesh,
        scratch_types=[
            pltpu.VMEM_SHARED((np_, D), jnp.float32),
            pltpu.VMEM((SUPER, CHUNK), jnp.int32),
            pltpu.VMEM((SUPER, CHUNK), jnp.int32),
            pltpu.VMEM((CHUNK, D), jnp.float32),
            pltpu.VMEM((CHUNK, D), jnp.float32),
            pltpu.SemaphoreType.DMA,
            pltpu.SemaphoreType.DMA,
        ],
        compiler_params=_SC_PARAMS)
    def kern(t0_hbm, t1_hbm, srcm_hbm, dstm_hbm, zrow_hbm, out_s,
             acc, srcbuf, dstbuf, rows0, rows1, sem0, sem1):
        c = lax.axis_index("c")
        s = lax.axis_index("s")
        pltpu.sync_copy(zrow_hbm, acc.at[pl.ds(s * rpt, rpt)])
        plsc.subcore_barrier()

        @pl.when(c == 0)
        def _():
            _edge_scatter_loop(t0_hbm, acc, srcm_hbm, dstm_hbm, srcbuf,
                               dstbuf, rows0, rows1, sem0, sem1,
                               s * nsupers, nsupers)

        @pl.when(c == 1)
        def _():
            _edge_scatter_loop(t1_hbm, acc, srcm_hbm, dstm_hbm, srcbuf,
                               dstbuf, rows0, rows1, sem0, sem1,
                               s * nsupers, nsupers)

        plsc.subcore_barrier()
        pltpu.sync_copy(acc.at[pl.ds(s * rpt, rpt)],
                        out_s.at[c].at[pl.ds(s * rpt, rpt)])

    return kern(t0, t1, srcm, dstm, zrow)


_BLK = 1024


def _tc_pre_mlp(xp, w1, b1, w2, b2, degp_t, np_):
    grid = np_ // _BLK

    def body(x_ref, w1_ref, b1_ref, w2_ref, b2_ref, degp_ref, h_ref,
             g_ref, scal_ref):
        t = jnp.dot(x_ref[...], w1_ref[...],
                    preferred_element_type=jnp.float32) + b1_ref[...]
        t = jnp.maximum(t, 0.0)
        h = jnp.dot(t, w2_ref[...],
                    preferred_element_type=jnp.float32) + b2_ref[...]
        h_ref[...] = h
        deg = jnp.sum(degp_ref[...], axis=1, keepdims=True)
        dis = lax.rsqrt(deg + 1.0)
        invd = 1.0 / jnp.maximum(deg, 1.0)
        g_ref[...] = h * dis
        scal_ref[...] = jnp.concatenate(
            [dis, dis * dis, invd, deg,
             jnp.zeros_like(deg), jnp.zeros_like(deg),
             jnp.zeros_like(deg), jnp.zeros_like(deg)], axis=1)

    return pl.pallas_call(
        body,
        grid=(grid,),
        in_specs=[
            pl.BlockSpec((_BLK, D), lambda i: (i, 0)),
            pl.BlockSpec((D, D), lambda i: (0, 0)),
            pl.BlockSpec((1, D), lambda i: (0, 0)),
            pl.BlockSpec((D, D), lambda i: (0, 0)),
            pl.BlockSpec((1, D), lambda i: (0, 0)),
            pl.BlockSpec((_BLK, NW), lambda i: (i, 0)),
        ],
        out_specs=[
            pl.BlockSpec((_BLK, D), lambda i: (i, 0)),
            pl.BlockSpec((_BLK, D), lambda i: (i, 0)),
            pl.BlockSpec((_BLK, 8), lambda i: (i, 0)),
        ],
        out_shape=[
            jax.ShapeDtypeStruct((np_, D), jnp.float32),
            jax.ShapeDtypeStruct((np_, D), jnp.float32),
            jax.ShapeDtypeStruct((np_, 8), jnp.float32),
        ],
    )(xp, w1, b1.reshape(1, D), w2, b2.reshape(1, D), degp_t)


def _tc_layer(h, s2, scal, wc, bc, lng, lnb, mix, np_, emit_g, add_skip):
    grid = np_ // _BLK

    def body(h_ref, sh_ref, sg_ref, scal_ref, wc_ref, bc_ref, lng_ref,
             lnb_ref, mix_ref, *out_refs):
        h_blk = h_ref[...]
        sh = sh_ref[0]
        sg = sg_ref[0]
        dis = scal_ref[:, 0:1]
        dis2 = scal_ref[:, 1:2]
        invd = scal_ref[:, 2:3]
        a_gcn = dis * sg + dis2 * h_blk
        mean = sh * invd
        z = jnp.concatenate([a_gcn, h_blk, mean], axis=1)
        u = jnp.dot(z, wc_ref[...],
                    preferred_element_type=jnp.float32) + bc_ref[...]
        m = jnp.mean(u, axis=1, keepdims=True)
        v = jnp.mean((u - m) * (u - m), axis=1, keepdims=True)
        hn = (u - m) * lax.rsqrt(v + 1e-5) * lng_ref[...] + lnb_ref[...]
        u2 = mix_ref[0, 0] * hn + mix_ref[0, 1] * u
        hnew = (mix_ref[0, 2] * jnp.maximum(u2, 0.0)
                + mix_ref[0, 3] * jnp.tanh(u2))
        if add_skip:
            out_refs[0][...] = h_blk + hnew
        else:
            out_refs[0][...] = hnew
        if emit_g:
            out_refs[1][...] = hnew * dis

    in_specs = [
        pl.BlockSpec((_BLK, D), lambda i: (i, 0)),
        pl.BlockSpec((1, _BLK, D), lambda i: (0, i, 0)),
        pl.BlockSpec((1, _BLK, D), lambda i: (1, i, 0)),
        pl.BlockSpec((_BLK, 8), lambda i: (i, 0)),
        pl.BlockSpec((3 * D, D), lambda i: (0, 0)),
        pl.BlockSpec((1, D), lambda i: (0, 0)),
        pl.BlockSpec((1, D), lambda i: (0, 0)),
        pl.BlockSpec((1, D), lambda i: (0, 0)),
        pl.BlockSpec((1, 8), lambda i: (0, 0)),
    ]
    out_specs = [pl.BlockSpec((_BLK, D), lambda i: (i, 0))]
    out_shape = [jax.ShapeDtypeStruct((np_, D), jnp.float32)]
    if emit_g:
        out_specs.append(pl.BlockSpec((_BLK, D), lambda i: (i, 0)))
        out_shape.append(jax.ShapeDtypeStruct((np_, D), jnp.float32))

    return pl.pallas_call(
        body,
        grid=(grid,),
        in_specs=in_specs,
        out_specs=out_specs,
        out_shape=out_shape,
    )(h, s2, s2, scal, wc, bc, lng, lnb, mix)


def _tc_pool_partial(part, batchp, np_):
    grid = np_ // _BLK

    def body(p_ref, b_ref, out_ref, pooled):
        i = pl.program_id(0)

        @pl.when(i == 0)
        def _():
            pooled[...] = jnp.zeros_like(pooled)

        col = lax.broadcasted_iota(jnp.int32, (_BLK, G_OUT), 1)
        m = (b_ref[...] == col).astype(jnp.float32)
        pooled[...] += lax.dot_general(
            m, p_ref[...], (((0,), (0,)), ((), ())),
            preferred_element_type=jnp.float32)

        @pl.when(i == grid - 1)
        def _():
            out_ref[...] = pooled[...]

    return pl.pallas_call(
        body,
        grid=(grid,),
        in_specs=[
            pl.BlockSpec((_BLK, D), lambda i: (i, 0)),
            pl.BlockSpec((_BLK, 1), lambda i: (i, 0)),
        ],
        out_specs=pl.BlockSpec((G_OUT, D), lambda i: (0, 0)),
        out_shape=jax.ShapeDtypeStruct((G_OUT, D), jnp.float32),
        scratch_shapes=[pltpu.VMEM((G_OUT, D), jnp.float32)],
    )(part, batchp)


def _tc_pool_post(h2, batchp, pool1, q1, qb1, q2, qb2, np_):
    grid = np_ // _BLK

    def body(h2_ref, b_ref, p1_ref, q1_ref, qb1_ref, q2_ref, qb2_ref,
             out_ref, pooled):
        i = pl.program_id(0)

        @pl.when(i == 0)
        def _():
            pooled[...] = p1_ref[...]

        col = lax.broadcasted_iota(jnp.int32, (_BLK, G_OUT), 1)
        m = (b_ref[...] == col).astype(jnp.float32)
        pooled[...] += lax.dot_general(
            m, h2_ref[...], (((0,), (0,)), ((), ())),
            preferred_element_type=jnp.float32)

        @pl.when(i == grid - 1)
        def _():
            t = jnp.dot(pooled[...], q1_ref[...],
                        preferred_element_type=jnp.float32) + qb1_ref[...]
            t = jnp.maximum(t, 0.0)
            out_ref[...] = jnp.dot(t, q2_ref[...],
                                   preferred_element_type=jnp.float32) \
                + qb2_ref[...]

    return pl.pallas_call(
        body,
        grid=(grid,),
        in_specs=[
            pl.BlockSpec((_BLK, D), lambda i: (i, 0)),
            pl.BlockSpec((_BLK, 1), lambda i: (i, 0)),
            pl.BlockSpec((G_OUT, D), lambda i: (0, 0)),
            pl.BlockSpec((D, D), lambda i: (0, 0)),
            pl.BlockSpec((1, D), lambda i: (0, 0)),
            pl.BlockSpec((D, D), lambda i: (0, 0)),
            pl.BlockSpec((1, D), lambda i: (0, 0)),
        ],
        out_specs=pl.BlockSpec((G_OUT, D), lambda i: (0, 0)),
        out_shape=jax.ShapeDtypeStruct((G_OUT, D), jnp.float32),
        scratch_shapes=[pltpu.VMEM((G_OUT, D), jnp.float32)],
    )(h2, batchp, pool1, q1, qb1.reshape(1, D), q2, qb2.reshape(1, D))


def _layer_weights(lp):
    ac = jax.nn.softmax(lp["alpha_conv"] / 1.0)
    an = jax.nn.softmax(lp["alpha_norm"] / 1.0)
    aa = jax.nn.softmax(lp["alpha_act"] / 1.0)
    wc = jnp.concatenate([ac[0] * lp["gcn"]["W"],
                          ac[1] * lp["sage"]["Ws"],
                          ac[1] * lp["sage"]["Wn"]], axis=0)
    bc = (ac[0] * lp["gcn"]["b"] + ac[1] * lp["sage"]["b"]).reshape(1, D)
    lng = lp["ln"]["g"].reshape(1, D)
    lnb = lp["ln"]["b"].reshape(1, D)
    mix = jnp.stack([an[0], an[1], aa[0], aa[1],
                     jnp.float32(0), jnp.float32(0),
                     jnp.float32(0), jnp.float32(0)]).reshape(1, 8)
    return wc, bc, lng, lnb, mix


def kernel(x, params, edge_index, batch):
    n, _ = x.shape
    e = edge_index.shape[1]
    np_ = _round_up(n + 1, _BLK)
    ep = _round_up(e, NW * SUPER * CHUNK)

    xp = jnp.pad(x, ((0, np_ - n), (0, 0)))
    pad_i = jnp.arange(ep - e, dtype=jnp.int32)
    pad_src = pad_i % n
    pad_dst = n + pad_i % (np_ - n)
    srcm = jnp.concatenate([edge_index[0], pad_src]).reshape(
        ep // CHUNK, CHUNK)
    dstm = jnp.concatenate([edge_index[1], pad_dst]).reshape(
        ep // CHUNK, CHUNK)
    batchp = jnp.pad(batch, (0, np_ - n),
                     constant_values=G_OUT).reshape(np_, 1)
    zrow = jnp.zeros((np_ // NS, D), jnp.float32)
    z1d = jnp.zeros((np_,), jnp.float32)

    degp = _sc_deg(dstm, z1d, np_, ep)
    pre = params["pre"]
    h, g1, scal = _tc_pre_mlp(xp, pre["W1"], pre["b1"], pre["W2"],
                              pre["b2"], degp.T, np_)

    s1 = _sc_scatter_two_tables(h, g1, srcm, dstm, zrow, np_, ep)
    wc1, bc1, lng1, lnb1, mix1 = _layer_weights(params["layers"][0])
    h1, g2 = _tc_layer(h, s1, scal, wc1, bc1, lng1, lnb1, mix1, np_,
                       emit_g=True, add_skip=False)

    s2 = _sc_scatter_two_tables(h1, g2, srcm, dstm, zrow, np_, ep)
    pool1 = _tc_pool_partial(h1, batchp, np_)
    wc2, bc2, lng2, lnb2, mix2 = _layer_weights(params["layers"][1])
    (h2,) = _tc_layer(h1, s2, scal, wc2, bc2, lng2, lnb2, mix2, np_,
                      emit_g=False, add_skip=False)

    post = params["post"]
    return _tc_pool_post(h2, batchp, pool1, post["W1"], post["b1"],
                         post["W2"], post["b2"], np_)


# speedup vs baseline: 1.0384x; 1.0384x over previous
import functools

import jax
import jax.numpy as jnp
from jax import lax
from jax.experimental import pallas as pl
from jax.experimental.pallas import tpu as pltpu
from jax.experimental.pallas import tpu_sc as plsc

NC, NS, LANES = 2, 16, 16
NW = NC * NS
D = 128
G_OUT = 128
CHUNK = 128
SUPER = 16

_SC_PARAMS = pltpu.CompilerParams(needs_layout_passes=False)


def _round_up(a, b):
    return (a + b - 1) // b * b


def _edge_scatter_loop(table, acc, srcm, dstm, srcbuf, dstbuf, rows0, rows1,
                       sem0, sem1, super0, nsupers):
    rows = (rows0, rows1)
    sems = (sem0, sem1)

    @pl.loop(0, nsupers)
    def _(jo):
        row0 = (super0 + jo) * SUPER
        pltpu.sync_copy(srcm.at[pl.ds(row0, SUPER)], srcbuf)
        pltpu.sync_copy(dstm.at[pl.ds(row0, SUPER)], dstbuf)
        pend = pltpu.async_copy(table.at[srcbuf.at[0]], rows[0], sems[0])
        for jj in range(SUPER):
            pend.wait()
            if jj + 1 < SUPER:
                pend = pltpu.async_copy(table.at[srcbuf.at[jj + 1]],
                                        rows[(jj + 1) % 2],
                                        sems[(jj + 1) % 2])
            pass


def _sc_deg(dstm, z1d, np_, ep):
    chunks = ep // CHUNK
    nsupers = chunks // (NW * SUPER)
    mesh = plsc.VectorSubcoreMesh(core_axis_name="c", subcore_axis_name="s",
                                  num_cores=NC, num_subcores=NS)

    @functools.partial(
        pl.kernel,
        out_type=jax.ShapeDtypeStruct((NW, np_), jnp.float32),
        mesh=mesh,
        scratch_types=[
            pltpu.VMEM((SUPER, CHUNK), jnp.int32),
            pltpu.VMEM((np_,), jnp.float32),
        ],
        compiler_params=_SC_PARAMS)
    def kern(dstm_hbm, z1d_hbm, out_deg, dstbuf, degv):
        c = lax.axis_index("c")
        s = lax.axis_index("s")
        w = c * NS + s
        pltpu.sync_copy(z1d_hbm, degv)

        @pl.loop(0, nsupers)
        def _(jo):
            row0 = (w * nsupers + jo) * SUPER
            pltpu.sync_copy(dstm_hbm.at[pl.ds(row0, SUPER)], dstbuf)
            for jj in range(SUPER):
                for q in range(CHUNK // LANES):
                    idx = dstbuf[jj, pl.ds(q * LANES, LANES)]
                    plsc.addupdate_scatter(
                        degv, [idx], jnp.ones((LANES,), jnp.float32))

        pltpu.sync_copy(degv, out_deg.at[w])

    return kern(dstm, z1d)


def _sc_scatter_two_tables(t0, t1, srcm, dstm, zrow, np_, ep):
    rpt = np_ // NS
    chunks = ep // CHUNK
    nsupers = chunks // (NS * SUPER)

    mesh = plsc.VectorSubcoreMesh(core_axis_name="c", subcore_axis_name="s",
                                  num_cores=NC, num_subcores=NS)

    @functools.partial(
        pl.kernel,
        out_type=jax.ShapeDtypeStruct((NC, np_, D), jnp.float32),
        mesh=mesh,
        scratch_types=[
            pltpu.VMEM_SHARED((np_, D), jnp.float32),
            pltpu.VMEM((SUPER, CHUNK), jnp.int32),
            pltpu.VMEM((SUPER, CHUNK), jnp.int32),
            pltpu.VMEM((CHUNK, D), jnp.float32),
            pltpu.VMEM((CHUNK, D), jnp.float32),
            pltpu.SemaphoreType.DMA,
            pltpu.SemaphoreType.DMA,
        ],
        compiler_params=_SC_PARAMS)
    def kern(t0_hbm, t1_hbm, srcm_hbm, dstm_hbm, zrow_hbm, out_s,
             acc, srcbuf, dstbuf, rows0, rows1, sem0, sem1):
        c = lax.axis_index("c")
        s = lax.axis_index("s")
        pltpu.sync_copy(zrow_hbm, acc.at[pl.ds(s * rpt, rpt)])
        plsc.subcore_barrier()

        @pl.when(c == 0)
        def _():
            _edge_scatter_loop(t0_hbm, acc, srcm_hbm, dstm_hbm, srcbuf,
                               dstbuf, rows0, rows1, sem0, sem1,
                               s * nsupers, nsupers)

        @pl.when(c == 1)
        def _():
            _edge_scatter_loop(t1_hbm, acc, srcm_hbm, dstm_hbm, srcbuf,
                               dstbuf, rows0, rows1, sem0, sem1,
                               s * nsupers, nsupers)

        plsc.subcore_barrier()
        pltpu.sync_copy(acc.at[pl.ds(s * rpt, rpt)],
                        out_s.at[c].at[pl.ds(s * rpt, rpt)])

    return kern(t0, t1, srcm, dstm, zrow)


_BLK = 1024


def _tc_pre_mlp(xp, w1, b1, w2, b2, degp_t, np_):
    grid = np_ // _BLK

    def body(x_ref, w1_ref, b1_ref, w2_ref, b2_ref, degp_ref, h_ref,
             g_ref, scal_ref):
        t = jnp.dot(x_ref[...], w1_ref[...],
                    preferred_element_type=jnp.float32) + b1_ref[...]
        t = jnp.maximum(t, 0.0)
        h = jnp.dot(t, w2_ref[...],
                    preferred_element_type=jnp.float32) + b2_ref[...]
        h_ref[...] = h
        deg = jnp.sum(degp_ref[...], axis=1, keepdims=True)
        dis = lax.rsqrt(deg + 1.0)
        invd = 1.0 / jnp.maximum(deg, 1.0)
        g_ref[...] = h * dis
        scal_ref[...] = jnp.concatenate(
            [dis, dis * dis, invd, deg,
             jnp.zeros_like(deg), jnp.zeros_like(deg),
             jnp.zeros_like(deg), jnp.zeros_like(deg)], axis=1)

    return pl.pallas_call(
        body,
        grid=(grid,),
        in_specs=[
            pl.BlockSpec((_BLK, D), lambda i: (i, 0)),
            pl.BlockSpec((D, D), lambda i: (0, 0)),
            pl.BlockSpec((1, D), lambda i: (0, 0)),
            pl.BlockSpec((D, D), lambda i: (0, 0)),
            pl.BlockSpec((1, D), lambda i: (0, 0)),
            pl.BlockSpec((_BLK, NW), lambda i: (i, 0)),
        ],
        out_specs=[
            pl.BlockSpec((_BLK, D), lambda i: (i, 0)),
            pl.BlockSpec((_BLK, D), lambda i: (i, 0)),
            pl.BlockSpec((_BLK, 8), lambda i: (i, 0)),
        ],
        out_shape=[
            jax.ShapeDtypeStruct((np_, D), jnp.float32),
            jax.ShapeDtypeStruct((np_, D), jnp.float32),
            jax.ShapeDtypeStruct((np_, 8), jnp.float32),
        ],
    )(xp, w1, b1.reshape(1, D), w2, b2.reshape(1, D), degp_t)


def _tc_layer(h, s2, scal, wc, bc, lng, lnb, mix, np_, emit_g, add_skip):
    grid = np_ // _BLK

    def body(h_ref, sh_ref, sg_ref, scal_ref, wc_ref, bc_ref, lng_ref,
             lnb_ref, mix_ref, *out_refs):
        h_blk = h_ref[...]
        sh = sh_ref[0]
        sg = sg_ref[0]
        dis = scal_ref[:, 0:1]
        dis2 = scal_ref[:, 1:2]
        invd = scal_ref[:, 2:3]
        a_gcn = dis * sg + dis2 * h_blk
        mean = sh * invd
        z = jnp.concatenate([a_gcn, h_blk, mean], axis=1)
        u = jnp.dot(z, wc_ref[...],
                    preferred_element_type=jnp.float32) + bc_ref[...]
        m = jnp.mean(u, axis=1, keepdims=True)
        v = jnp.mean((u - m) * (u - m), axis=1, keepdims=True)
        hn = (u - m) * lax.rsqrt(v + 1e-5) * lng_ref[...] + lnb_ref[...]
        u2 = mix_ref[0, 0] * hn + mix_ref[0, 1] * u
        hnew = (mix_ref[0, 2] * jnp.maximum(u2, 0.0)
                + mix_ref[0, 3] * jnp.tanh(u2))
        if add_skip:
            out_refs[0][...] = h_blk + hnew
        else:
            out_refs[0][...] = hnew
        if emit_g:
            out_refs[1][...] = hnew * dis

    in_specs = [
        pl.BlockSpec((_BLK, D), lambda i: (i, 0)),
        pl.BlockSpec((1, _BLK, D), lambda i: (0, i, 0)),
        pl.BlockSpec((1, _BLK, D), lambda i: (1, i, 0)),
        pl.BlockSpec((_BLK, 8), lambda i: (i, 0)),
        pl.BlockSpec((3 * D, D), lambda i: (0, 0)),
        pl.BlockSpec((1, D), lambda i: (0, 0)),
        pl.BlockSpec((1, D), lambda i: (0, 0)),
        pl.BlockSpec((1, D), lambda i: (0, 0)),
        pl.BlockSpec((1, 8), lambda i: (0, 0)),
    ]
    out_specs = [pl.BlockSpec((_BLK, D), lambda i: (i, 0))]
    out_shape = [jax.ShapeDtypeStruct((np_, D), jnp.float32)]
    if emit_g:
        out_specs.append(pl.BlockSpec((_BLK, D), lambda i: (i, 0)))
        out_shape.append(jax.ShapeDtypeStruct((np_, D), jnp.float32))

    return pl.pallas_call(
        body,
        grid=(grid,),
        in_specs=in_specs,
        out_specs=out_specs,
        out_shape=out_shape,
    )(h, s2, s2, scal, wc, bc, lng, lnb, mix)


def _tc_pool_partial(part, batchp, np_):
    grid = np_ // _BLK

    def body(p_ref, b_ref, out_ref, pooled):
        i = pl.program_id(0)

        @pl.when(i == 0)
        def _():
            pooled[...] = jnp.zeros_like(pooled)

        col = lax.broadcasted_iota(jnp.int32, (_BLK, G_OUT), 1)
        m = (b_ref[...] == col).astype(jnp.float32)
        pooled[...] += lax.dot_general(
            m, p_ref[...], (((0,), (0,)), ((), ())),
            preferred_element_type=jnp.float32)

        @pl.when(i == grid - 1)
        def _():
            out_ref[...] = pooled[...]

    return pl.pallas_call(
        body,
        grid=(grid,),
        in_specs=[
            pl.BlockSpec((_BLK, D), lambda i: (i, 0)),
            pl.BlockSpec((_BLK, 1), lambda i: (i, 0)),
        ],
        out_specs=pl.BlockSpec((G_OUT, D), lambda i: (0, 0)),
        out_shape=jax.ShapeDtypeStruct((G_OUT, D), jnp.float32),
        scratch_shapes=[pltpu.VMEM((G_OUT, D), jnp.float32)],
    )(part, batchp)


def _tc_pool_post(h2, batchp, pool1, q1, qb1, q2, qb2, np_):
    grid = np_ // _BLK

    def body(h2_ref, b_ref, p1_ref, q1_ref, qb1_ref, q2_ref, qb2_ref,
             out_ref, pooled):
        i = pl.program_id(0)

        @pl.when(i == 0)
        def _():
            pooled[...] = p1_ref[...]

        col = lax.broadcasted_iota(jnp.int32, (_BLK, G_OUT), 1)
        m = (b_ref[...] == col).astype(jnp.float32)
        pooled[...] += lax.dot_general(
            m, h2_ref[...], (((0,), (0,)), ((), ())),
            preferred_element_type=jnp.float32)

        @pl.when(i == grid - 1)
        def _():
            t = jnp.dot(pooled[...], q1_ref[...],
                        preferred_element_type=jnp.float32) + qb1_ref[...]
            t = jnp.maximum(t, 0.0)
            out_ref[...] = jnp.dot(t, q2_ref[...],
                                   preferred_element_type=jnp.float32) \
                + qb2_ref[...]

    return pl.pallas_call(
        body,
        grid=(grid,),
        in_specs=[
            pl.BlockSpec((_BLK, D), lambda i: (i, 0)),
            pl.BlockSpec((_BLK, 1), lambda i: (i, 0)),
            pl.BlockSpec((G_OUT, D), lambda i: (0, 0)),
            pl.BlockSpec((D, D), lambda i: (0, 0)),
            pl.BlockSpec((1, D), lambda i: (0, 0)),
            pl.BlockSpec((D, D), lambda i: (0, 0)),
            pl.BlockSpec((1, D), lambda i: (0, 0)),
        ],
        out_specs=pl.BlockSpec((G_OUT, D), lambda i: (0, 0)),
        out_shape=jax.ShapeDtypeStruct((G_OUT, D), jnp.float32),
        scratch_shapes=[pltpu.VMEM((G_OUT, D), jnp.float32)],
    )(h2, batchp, pool1, q1, qb1.reshape(1, D), q2, qb2.reshape(1, D))


def _layer_weights(lp):
    ac = jax.nn.softmax(lp["alpha_conv"] / 1.0)
    an = jax.nn.softmax(lp["alpha_norm"] / 1.0)
    aa = jax.nn.softmax(lp["alpha_act"] / 1.0)
    wc = jnp.concatenate([ac[0] * lp["gcn"]["W"],
                          ac[1] * lp["sage"]["Ws"],
                          ac[1] * lp["sage"]["Wn"]], axis=0)
    bc = (ac[0] * lp["gcn"]["b"] + ac[1] * lp["sage"]["b"]).reshape(1, D)
    lng = lp["ln"]["g"].reshape(1, D)
    lnb = lp["ln"]["b"].reshape(1, D)
    mix = jnp.stack([an[0], an[1], aa[0], aa[1],
                     jnp.float32(0), jnp.float32(0),
                     jnp.float32(0), jnp.float32(0)]).reshape(1, 8)
    return wc, bc, lng, lnb, mix


def kernel(x, params, edge_index, batch):
    n, _ = x.shape
    e = edge_index.shape[1]
    np_ = _round_up(n + 1, _BLK)
    ep = _round_up(e, NW * SUPER * CHUNK)

    xp = jnp.pad(x, ((0, np_ - n), (0, 0)))
    pad_i = jnp.arange(ep - e, dtype=jnp.int32)
    pad_src = pad_i % n
    pad_dst = n + pad_i % (np_ - n)
    srcm = jnp.concatenate([edge_index[0], pad_src]).reshape(
        ep // CHUNK, CHUNK)
    dstm = jnp.concatenate([edge_index[1], pad_dst]).reshape(
        ep // CHUNK, CHUNK)
    batchp = jnp.pad(batch, (0, np_ - n),
                     constant_values=G_OUT).reshape(np_, 1)
    zrow = jnp.zeros((np_ // NS, D), jnp.float32)
    z1d = jnp.zeros((np_,), jnp.float32)

    degp = _sc_deg(dstm, z1d, np_, ep)
    pre = params["pre"]
    h, g1, scal = _tc_pre_mlp(xp, pre["W1"], pre["b1"], pre["W2"],
                              pre["b2"], degp.T, np_)

    s1 = _sc_scatter_two_tables(h, g1, srcm, dstm, zrow, np_, ep)
    wc1, bc1, lng1, lnb1, mix1 = _layer_weights(params["layers"][0])
    h1, g2 = _tc_layer(h, s1, scal, wc1, bc1, lng1, lnb1, mix1, np_,
                       emit_g=True, add_skip=False)

    s2 = _sc_scatter_two_tables(h1, g2, srcm, dstm, zrow, np_, ep)
    pool1 = _tc_pool_partial(h1, batchp, np_)
    wc2, bc2, lng2, lnb2, mix2 = _layer_weights(params["layers"][1])
    (h2,) = _tc_layer(h1, s2, scal, wc2, bc2, lng2, lnb2, mix2, np_,
                      emit_g=False, add_skip=False)

    post = params["post"]
    return _tc_pool_post(h2, batchp, pool1, post["W1"], post["b1"],
                         post["W2"], post["b2"], np_)


# speedup vs baseline: 1.1614x; 1.1185x over previous
import functools

import jax
import jax.numpy as jnp
from jax import lax
from jax.experimental import pallas as pl
from jax.experimental.pallas import tpu as pltpu
from jax.experimental.pallas import tpu_sc as plsc

NC, NS, LANES = 2, 16, 16
NW = NC * NS
D = 128
G_OUT = 128
CHUNK = 112
SUPER = 16
SUPER_DEG = 16

_SC_PARAMS = pltpu.CompilerParams(needs_layout_passes=False)


def _round_up(a, b):
    return (a + b - 1) // b * b


_NBUF = 3
_DEPTH = 2


def _edge_scatter_loop(table, acc, srcm, dstm, srcbuf, dstbuf, rows, sems,
                       super0, nsupers):
    @pl.loop(0, nsupers)
    def _(jo):
        row0 = (super0 + jo) * SUPER
        pltpu.sync_copy(srcm.at[pl.ds(row0, SUPER)], srcbuf)
        pltpu.sync_copy(dstm.at[pl.ds(row0, SUPER)], dstbuf)
        pend = [None] * _NBUF

        def fire(k):
            pend[k % _NBUF] = pltpu.async_copy(
                table.at[srcbuf.at[k]], rows[k % _NBUF], sems[k % _NBUF])

        for k in range(min(_DEPTH, SUPER)):
            fire(k)
        for jj in range(SUPER):
            if jj + _DEPTH < SUPER:
                fire(jj + _DEPTH)
            pend[jj % _NBUF].wait()
            pltpu.sync_copy(rows[jj % _NBUF], acc.at[dstbuf.at[jj]],
                            add=True)


def _sc_deg(dstm, z1d, np_, ep):
    chunks = ep // CHUNK
    nsupers = chunks // (NW * SUPER_DEG)
    mesh = plsc.VectorSubcoreMesh(core_axis_name="c", subcore_axis_name="s",
                                  num_cores=NC, num_subcores=NS)

    @functools.partial(
        pl.kernel,
        out_type=jax.ShapeDtypeStruct((NW, np_), jnp.float32),
        mesh=mesh,
        scratch_types=[
            pltpu.VMEM((SUPER_DEG, CHUNK), jnp.int32),
            pltpu.VMEM((np_,), jnp.float32),
        ],
        compiler_params=_SC_PARAMS)
    def kern(dstm_hbm, z1d_hbm, out_deg, dstbuf, degv):
        c = lax.axis_index("c")
        s = lax.axis_index("s")
        w = c * NS + s
        pltpu.sync_copy(z1d_hbm, degv)

        @pl.loop(0, nsupers)
        def _(jo):
            row0 = (w * nsupers + jo) * SUPER_DEG
            pltpu.sync_copy(dstm_hbm.at[pl.ds(row0, SUPER_DEG)], dstbuf)
            for jj in range(SUPER_DEG):
                for q in range(CHUNK // LANES):
                    idx = dstbuf[jj, pl.ds(q * LANES, LANES)]
                    plsc.addupdate_scatter(
                        degv, [idx], jnp.ones((LANES,), jnp.float32))

        pltpu.sync_copy(degv, out_deg.at[w])

    return kern(dstm, z1d)


def _sc_scatter_two_tables(t0, t1, srcm, dstm, zrow, np_, ep):
    rpt = np_ // NS
    chunks = ep // CHUNK
    nsupers = chunks // (NS * SUPER)

    mesh = plsc.VectorSubcoreMesh(core_axis_name="c", subcore_axis_name="s",
                                  num_cores=NC, num_subcores=NS)

    @functools.partial(
        pl.kernel,
        out_type=jax.ShapeDtypeStruct((NC, np_, D), jnp.float32),
        mesh=mesh,
        scratch_types=[
            pltpu.VMEM_SHARED((np_, D), jnp.float32),
            pltpu.VMEM((SUPER, CHUNK), jnp.int32),
            pltpu.VMEM((SUPER, CHUNK), jnp.int32),
        ] + [pltpu.VMEM((CHUNK, D), jnp.float32)] * _NBUF
          + [pltpu.SemaphoreType.DMA] * _NBUF,
        compiler_params=_SC_PARAMS)
    def kern(t0_hbm, t1_hbm, srcm_hbm, dstm_hbm, zrow_hbm, out_s,
             acc, srcbuf, dstbuf, *rest):
        rows = rest[:_NBUF]
        sems = rest[_NBUF:2 * _NBUF]
        c = lax.axis_index("c")
        s = lax.axis_index("s")
        pltpu.sync_copy(zrow_hbm, acc.at[pl.ds(s * rpt, rpt)])
        plsc.subcore_barrier()

        @pl.when(c == 0)
        def _():
            _edge_scatter_loop(t0_hbm, acc, srcm_hbm, dstm_hbm, srcbuf,
                               dstbuf, rows, sems, s * nsupers, nsupers)

        @pl.when(c == 1)
        def _():
            _edge_scatter_loop(t1_hbm, acc, srcm_hbm, dstm_hbm, srcbuf,
                               dstbuf, rows, sems, s * nsupers, nsupers)

        plsc.subcore_barrier()
        pltpu.sync_copy(acc.at[pl.ds(s * rpt, rpt)],
                        out_s.at[c].at[pl.ds(s * rpt, rpt)])

    return kern(t0, t1, srcm, dstm, zrow)


_BLK = 1024


def _tc_pre_mlp(xp, w1, b1, w2, b2, degp_t, np_):
    grid = np_ // _BLK

    def body(x_ref, w1_ref, b1_ref, w2_ref, b2_ref, degp_ref, h_ref,
             g_ref, scal_ref):
        t = jnp.dot(x_ref[...], w1_ref[...],
                    preferred_element_type=jnp.float32) + b1_ref[...]
        t = jnp.maximum(t, 0.0)
        h = jnp.dot(t, w2_ref[...],
                    preferred_element_type=jnp.float32) + b2_ref[...]
        h_ref[...] = h
        deg = jnp.sum(degp_ref[...], axis=1, keepdims=True)
        dis = lax.rsqrt(deg + 1.0)
        invd = 1.0 / jnp.maximum(deg, 1.0)
        g_ref[...] = h * dis
        scal_ref[...] = jnp.concatenate(
            [dis, dis * dis, invd, deg,
             jnp.zeros_like(deg), jnp.zeros_like(deg),
             jnp.zeros_like(deg), jnp.zeros_like(deg)], axis=1)

    return pl.pallas_call(
        body,
        grid=(grid,),
        in_specs=[
            pl.BlockSpec((_BLK, D), lambda i: (i, 0)),
            pl.BlockSpec((D, D), lambda i: (0, 0)),
            pl.BlockSpec((1, D), lambda i: (0, 0)),
            pl.BlockSpec((D, D), lambda i: (0, 0)),
            pl.BlockSpec((1, D), lambda i: (0, 0)),
            pl.BlockSpec((_BLK, NW), lambda i: (i, 0)),
        ],
        out_specs=[
            pl.BlockSpec((_BLK, D), lambda i: (i, 0)),
            pl.BlockSpec((_BLK, D), lambda i: (i, 0)),
            pl.BlockSpec((_BLK, 8), lambda i: (i, 0)),
        ],
        out_shape=[
            jax.ShapeDtypeStruct((np_, D), jnp.float32),
            jax.ShapeDtypeStruct((np_, D), jnp.float32),
            jax.ShapeDtypeStruct((np_, 8), jnp.float32),
        ],
    )(xp, w1, b1.reshape(1, D), w2, b2.reshape(1, D), degp_t)


def _tc_layer(h, s2, scal, wc, bc, lng, lnb, mix, np_, emit_g, add_skip):
    grid = np_ // _BLK

    def body(h_ref, sh_ref, sg_ref, scal_ref, wc_ref, bc_ref, lng_ref,
             lnb_ref, mix_ref, *out_refs):
        h_blk = h_ref[...]
        sh = sh_ref[0]
        sg = sg_ref[0]
        dis = scal_ref[:, 0:1]
        dis2 = scal_ref[:, 1:2]
        invd = scal_ref[:, 2:3]
        a_gcn = dis * sg + dis2 * h_blk
        mean = sh * invd
        z = jnp.concatenate([a_gcn, h_blk, mean], axis=1)
        u = jnp.dot(z, wc_ref[...],
                    preferred_element_type=jnp.float32) + bc_ref[...]
        m = jnp.mean(u, axis=1, keepdims=True)
        v = jnp.mean((u - m) * (u - m), axis=1, keepdims=True)
        hn = (u - m) * lax.rsqrt(v + 1e-5) * lng_ref[...] + lnb_ref[...]
        u2 = mix_ref[0, 0] * hn + mix_ref[0, 1] * u
        hnew = (mix_ref[0, 2] * jnp.maximum(u2, 0.0)
                + mix_ref[0, 3] * jnp.tanh(u2))
        if add_skip:
            out_refs[0][...] = h_blk + hnew
        else:
            out_refs[0][...] = hnew
        if emit_g:
            out_refs[1][...] = hnew * dis

    in_specs = [
        pl.BlockSpec((_BLK, D), lambda i: (i, 0)),
        pl.BlockSpec((1, _BLK, D), lambda i: (0, i, 0)),
        pl.BlockSpec((1, _BLK, D), lambda i: (1, i, 0)),
        pl.BlockSpec((_BLK, 8), lambda i: (i, 0)),
        pl.BlockSpec((3 * D, D), lambda i: (0, 0)),
        pl.BlockSpec((1, D), lambda i: (0, 0)),
        pl.BlockSpec((1, D), lambda i: (0, 0)),
        pl.BlockSpec((1, D), lambda i: (0, 0)),
        pl.BlockSpec((1, 8), lambda i: (0, 0)),
    ]
    out_specs = [pl.BlockSpec((_BLK, D), lambda i: (i, 0))]
    out_shape = [jax.ShapeDtypeStruct((np_, D), jnp.float32)]
    if emit_g:
        out_specs.append(pl.BlockSpec((_BLK, D), lambda i: (i, 0)))
        out_shape.append(jax.ShapeDtypeStruct((np_, D), jnp.float32))

    return pl.pallas_call(
        body,
        grid=(grid,),
        in_specs=in_specs,
        out_specs=out_specs,
        out_shape=out_shape,
    )(h, s2, s2, scal, wc, bc, lng, lnb, mix)


def _tc_pool_partial(part, batchp, np_):
    grid = np_ // _BLK

    def body(p_ref, b_ref, out_ref, pooled):
        i = pl.program_id(0)

        @pl.when(i == 0)
        def _():
            pooled[...] = jnp.zeros_like(pooled)

        col = lax.broadcasted_iota(jnp.int32, (_BLK, G_OUT), 1)
        m = (b_ref[...] == col).astype(jnp.float32)
        pooled[...] += lax.dot_general(
            m, p_ref[...], (((0,), (0,)), ((), ())),
            preferred_element_type=jnp.float32)

        @pl.when(i == grid - 1)
        def _():
            out_ref[...] = pooled[...]

    return pl.pallas_call(
        body,
        grid=(grid,),
        in_specs=[
            pl.BlockSpec((_BLK, D), lambda i: (i, 0)),
            pl.BlockSpec((_BLK, 1), lambda i: (i, 0)),
        ],
        out_specs=pl.BlockSpec((G_OUT, D), lambda i: (0, 0)),
        out_shape=jax.ShapeDtypeStruct((G_OUT, D), jnp.float32),
        scratch_shapes=[pltpu.VMEM((G_OUT, D), jnp.float32)],
    )(part, batchp)


def _tc_pool_post(h2, batchp, pool1, q1, qb1, q2, qb2, np_):
    grid = np_ // _BLK

    def body(h2_ref, b_ref, p1_ref, q1_ref, qb1_ref, q2_ref, qb2_ref,
             out_ref, pooled):
        i = pl.program_id(0)

        @pl.when(i == 0)
        def _():
            pooled[...] = p1_ref[...]

        col = lax.broadcasted_iota(jnp.int32, (_BLK, G_OUT), 1)
        m = (b_ref[...] == col).astype(jnp.float32)
        pooled[...] += lax.dot_general(
            m, h2_ref[...], (((0,), (0,)), ((), ())),
            preferred_element_type=jnp.float32)

        @pl.when(i == grid - 1)
        def _():
            t = jnp.dot(pooled[...], q1_ref[...],
                        preferred_element_type=jnp.float32) + qb1_ref[...]
            t = jnp.maximum(t, 0.0)
            out_ref[...] = jnp.dot(t, q2_ref[...],
                                   preferred_element_type=jnp.float32) \
                + qb2_ref[...]

    return pl.pallas_call(
        body,
        grid=(grid,),
        in_specs=[
            pl.BlockSpec((_BLK, D), lambda i: (i, 0)),
            pl.BlockSpec((_BLK, 1), lambda i: (i, 0)),
            pl.BlockSpec((G_OUT, D), lambda i: (0, 0)),
            pl.BlockSpec((D, D), lambda i: (0, 0)),
            pl.BlockSpec((1, D), lambda i: (0, 0)),
            pl.BlockSpec((D, D), lambda i: (0, 0)),
            pl.BlockSpec((1, D), lambda i: (0, 0)),
        ],
        out_specs=pl.BlockSpec((G_OUT, D), lambda i: (0, 0)),
        out_shape=jax.ShapeDtypeStruct((G_OUT, D), jnp.float32),
        scratch_shapes=[pltpu.VMEM((G_OUT, D), jnp.float32)],
    )(h2, batchp, pool1, q1, qb1.reshape(1, D), q2, qb2.reshape(1, D))


def _layer_weights(lp):
    ac = jax.nn.softmax(lp["alpha_conv"] / 1.0)
    an = jax.nn.softmax(lp["alpha_norm"] / 1.0)
    aa = jax.nn.softmax(lp["alpha_act"] / 1.0)
    wc = jnp.concatenate([ac[0] * lp["gcn"]["W"],
                          ac[1] * lp["sage"]["Ws"],
                          ac[1] * lp["sage"]["Wn"]], axis=0)
    bc = (ac[0] * lp["gcn"]["b"] + ac[1] * lp["sage"]["b"]).reshape(1, D)
    lng = lp["ln"]["g"].reshape(1, D)
    lnb = lp["ln"]["b"].reshape(1, D)
    mix = jnp.stack([an[0], an[1], aa[0], aa[1],
                     jnp.float32(0), jnp.float32(0),
                     jnp.float32(0), jnp.float32(0)]).reshape(1, 8)
    return wc, bc, lng, lnb, mix


def kernel(x, params, edge_index, batch):
    n, _ = x.shape
    e = edge_index.shape[1]
    np_ = _round_up(n + 1, _BLK)
    ep = _round_up(e, NW * SUPER * CHUNK)

    xp = jnp.pad(x, ((0, np_ - n), (0, 0)))
    pad_i = jnp.arange(ep - e, dtype=jnp.int32)
    pad_src = pad_i % n
    pad_dst = n + pad_i % (np_ - n)
    srcm = jnp.concatenate([edge_index[0], pad_src]).reshape(
        ep // CHUNK, CHUNK)
    dstm = jnp.concatenate([edge_index[1], pad_dst]).reshape(
        ep // CHUNK, CHUNK)
    batchp = jnp.pad(batch, (0, np_ - n),
                     constant_values=G_OUT).reshape(np_, 1)
    zrow = jnp.zeros((np_ // NS, D), jnp.float32)
    z1d = jnp.zeros((np_,), jnp.float32)

    degp = _sc_deg(dstm, z1d, np_, ep)
    pre = params["pre"]
    h, g1, scal = _tc_pre_mlp(xp, pre["W1"], pre["b1"], pre["W2"],
                              pre["b2"], degp.T, np_)

    s1 = _sc_scatter_two_tables(h, g1, srcm, dstm, zrow, np_, ep)
    wc1, bc1, lng1, lnb1, mix1 = _layer_weights(params["layers"][0])
    h1, g2 = _tc_layer(h, s1, scal, wc1, bc1, lng1, lnb1, mix1, np_,
                       emit_g=True, add_skip=False)

    s2 = _sc_scatter_two_tables(h1, g2, srcm, dstm, zrow, np_, ep)
    pool1 = _tc_pool_partial(h1, batchp, np_)
    wc2, bc2, lng2, lnb2, mix2 = _layer_weights(params["layers"][1])
    (h2,) = _tc_layer(h1, s2, scal, wc2, bc2, lng2, lnb2, mix2, np_,
                      emit_g=False, add_skip=False)

    post = params["post"]
    return _tc_pool_post(h2, batchp, pool1, post["W1"], post["b1"],
                         post["W2"], post["b2"], np_)


# speedup vs baseline: 1.2107x; 1.0424x over previous
import functools

import jax
import jax.numpy as jnp
from jax import lax
from jax.experimental import pallas as pl
from jax.experimental.pallas import tpu as pltpu
from jax.experimental.pallas import tpu_sc as plsc

NC, NS, LANES = 2, 16, 16
NW = NC * NS
D = 128
G_OUT = 128
CHUNK = 112
SUPER = 24
SUPER_DEG = 16

_SC_PARAMS = pltpu.CompilerParams(needs_layout_passes=False)


def _round_up(a, b):
    return (a + b - 1) // b * b


_NBUF = 3
_DEPTH = 2


def _edge_scatter_loop(table, acc, srcm, dstm, srcbuf, dstbuf, rows, sems,
                       super0, nsupers):
    ssem = sems[_NBUF]

    @pl.loop(0, nsupers)
    def _(jo):
        row0 = (super0 + jo) * SUPER
        pltpu.sync_copy(srcm.at[pl.ds(row0, SUPER)], srcbuf)
        pltpu.sync_copy(dstm.at[pl.ds(row0, SUPER)], dstbuf)
        pend = [None] * _NBUF
        pend_sc = [None]

        def fire(k):
            pend[k % _NBUF] = pltpu.async_copy(
                table.at[srcbuf.at[k]], rows[k % _NBUF], sems[k % _NBUF])

        for k in range(min(_DEPTH, SUPER)):
            fire(k)
        for jj in range(SUPER):
            if pend_sc[0] is not None:
                pend_sc[0].wait()
            if jj + _DEPTH < SUPER:
                fire(jj + _DEPTH)
            pend[jj % _NBUF].wait()
            if jj < SUPER - 1:
                pend_sc[0] = pltpu.async_copy(
                    rows[jj % _NBUF], acc.at[dstbuf.at[jj]], ssem, add=True)
            else:
                pltpu.sync_copy(rows[jj % _NBUF], acc.at[dstbuf.at[jj]],
                                add=True)


def _sc_deg(dstm, z1d, np_, ep):
    chunks = ep // CHUNK
    nsupers = chunks // (NW * SUPER_DEG)
    mesh = plsc.VectorSubcoreMesh(core_axis_name="c", subcore_axis_name="s",
                                  num_cores=NC, num_subcores=NS)

    @functools.partial(
        pl.kernel,
        out_type=jax.ShapeDtypeStruct((NW, np_), jnp.float32),
        mesh=mesh,
        scratch_types=[
            pltpu.VMEM((SUPER_DEG, CHUNK), jnp.int32),
            pltpu.VMEM((np_,), jnp.float32),
        ],
        compiler_params=_SC_PARAMS)
    def kern(dstm_hbm, z1d_hbm, out_deg, dstbuf, degv):
        c = lax.axis_index("c")
        s = lax.axis_index("s")
        w = c * NS + s
        pltpu.sync_copy(z1d_hbm, degv)

        @pl.loop(0, nsupers)
        def _(jo):
            row0 = (w * nsupers + jo) * SUPER_DEG
            pltpu.sync_copy(dstm_hbm.at[pl.ds(row0, SUPER_DEG)], dstbuf)
            for jj in range(SUPER_DEG):
                for q in range(CHUNK // LANES):
                    idx = dstbuf[jj, pl.ds(q * LANES, LANES)]
                    plsc.addupdate_scatter(
                        degv, [idx], jnp.ones((LANES,), jnp.float32))

        pltpu.sync_copy(degv, out_deg.at[w])

    return kern(dstm, z1d)


def _sc_scatter_two_tables(t0, t1, srcm, dstm, zrow, np_, ep):
    rpt = np_ // NS
    chunks = ep // CHUNK
    nsupers = chunks // (NS * SUPER)

    mesh = plsc.VectorSubcoreMesh(core_axis_name="c", subcore_axis_name="s",
                                  num_cores=NC, num_subcores=NS)

    @functools.partial(
        pl.kernel,
        out_type=jax.ShapeDtypeStruct((NC, np_, D), jnp.float32),
        mesh=mesh,
        scratch_types=[
            pltpu.VMEM_SHARED((np_, D), jnp.float32),
            pltpu.VMEM((SUPER, CHUNK), jnp.int32),
            pltpu.VMEM((SUPER, CHUNK), jnp.int32),
        ] + [pltpu.VMEM((CHUNK, D), jnp.float32)] * _NBUF
          + [pltpu.SemaphoreType.DMA] * (_NBUF + 1),
        compiler_params=_SC_PARAMS)
    def kern(t0_hbm, t1_hbm, srcm_hbm, dstm_hbm, zrow_hbm, out_s,
             acc, srcbuf, dstbuf, *rest):
        rows = rest[:_NBUF]
        sems = rest[_NBUF:2 * _NBUF + 1]
        c = lax.axis_index("c")
        s = lax.axis_index("s")
        pltpu.sync_copy(zrow_hbm, acc.at[pl.ds(s * rpt, rpt)])
        plsc.subcore_barrier()

        @pl.when(c == 0)
        def _():
            _edge_scatter_loop(t0_hbm, acc, srcm_hbm, dstm_hbm, srcbuf,
                               dstbuf, rows, sems, s * nsupers, nsupers)

        @pl.when(c == 1)
        def _():
            _edge_scatter_loop(t1_hbm, acc, srcm_hbm, dstm_hbm, srcbuf,
                               dstbuf, rows, sems, s * nsupers, nsupers)

        plsc.subcore_barrier()
        pltpu.sync_copy(acc.at[pl.ds(s * rpt, rpt)],
                        out_s.at[c].at[pl.ds(s * rpt, rpt)])

    return kern(t0, t1, srcm, dstm, zrow)


_BLK = 1024


def _tc_pre_mlp(xp, w1, b1, w2, b2, degp_t, np_):
    grid = np_ // _BLK

    def body(x_ref, w1_ref, b1_ref, w2_ref, b2_ref, degp_ref, h_ref,
             g_ref, scal_ref):
        t = jnp.dot(x_ref[...], w1_ref[...],
                    preferred_element_type=jnp.float32) + b1_ref[...]
        t = jnp.maximum(t, 0.0)
        h = jnp.dot(t, w2_ref[...],
                    preferred_element_type=jnp.float32) + b2_ref[...]
        h_ref[...] = h
        deg = jnp.sum(degp_ref[...], axis=1, keepdims=True)
        dis = lax.rsqrt(deg + 1.0)
        invd = 1.0 / jnp.maximum(deg, 1.0)
        g_ref[...] = h * dis
        scal_ref[...] = jnp.concatenate(
            [dis, dis * dis, invd, deg,
             jnp.zeros_like(deg), jnp.zeros_like(deg),
             jnp.zeros_like(deg), jnp.zeros_like(deg)], axis=1)

    return pl.pallas_call(
        body,
        grid=(grid,),
        in_specs=[
            pl.BlockSpec((_BLK, D), lambda i: (i, 0)),
            pl.BlockSpec((D, D), lambda i: (0, 0)),
            pl.BlockSpec((1, D), lambda i: (0, 0)),
            pl.BlockSpec((D, D), lambda i: (0, 0)),
            pl.BlockSpec((1, D), lambda i: (0, 0)),
            pl.BlockSpec((_BLK, NW), lambda i: (i, 0)),
        ],
        out_specs=[
            pl.BlockSpec((_BLK, D), lambda i: (i, 0)),
            pl.BlockSpec((_BLK, D), lambda i: (i, 0)),
            pl.BlockSpec((_BLK, 8), lambda i: (i, 0)),
        ],
        out_shape=[
            jax.ShapeDtypeStruct((np_, D), jnp.float32),
            jax.ShapeDtypeStruct((np_, D), jnp.float32),
            jax.ShapeDtypeStruct((np_, 8), jnp.float32),
        ],
    )(xp, w1, b1.reshape(1, D), w2, b2.reshape(1, D), degp_t)


def _tc_layer(h, s2, scal, wc, bc, lng, lnb, mix, np_, emit_g, add_skip):
    grid = np_ // _BLK

    def body(h_ref, sh_ref, sg_ref, scal_ref, wc_ref, bc_ref, lng_ref,
             lnb_ref, mix_ref, *out_refs):
        h_blk = h_ref[...]
        sh = sh_ref[0]
        sg = sg_ref[0]
        dis = scal_ref[:, 0:1]
        dis2 = scal_ref[:, 1:2]
        invd = scal_ref[:, 2:3]
        a_gcn = dis * sg + dis2 * h_blk
        mean = sh * invd
        z = jnp.concatenate([a_gcn, h_blk, mean], axis=1)
        u = jnp.dot(z, wc_ref[...],
                    preferred_element_type=jnp.float32) + bc_ref[...]
        m = jnp.mean(u, axis=1, keepdims=True)
        v = jnp.mean((u - m) * (u - m), axis=1, keepdims=True)
        hn = (u - m) * lax.rsqrt(v + 1e-5) * lng_ref[...] + lnb_ref[...]
        u2 = mix_ref[0, 0] * hn + mix_ref[0, 1] * u
        hnew = (mix_ref[0, 2] * jnp.maximum(u2, 0.0)
                + mix_ref[0, 3] * jnp.tanh(u2))
        if add_skip:
            out_refs[0][...] = h_blk + hnew
        else:
            out_refs[0][...] = hnew
        if emit_g:
            out_refs[1][...] = hnew * dis

    in_specs = [
        pl.BlockSpec((_BLK, D), lambda i: (i, 0)),
        pl.BlockSpec((1, _BLK, D), lambda i: (0, i, 0)),
        pl.BlockSpec((1, _BLK, D), lambda i: (1, i, 0)),
        pl.BlockSpec((_BLK, 8), lambda i: (i, 0)),
        pl.BlockSpec((3 * D, D), lambda i: (0, 0)),
        pl.BlockSpec((1, D), lambda i: (0, 0)),
        pl.BlockSpec((1, D), lambda i: (0, 0)),
        pl.BlockSpec((1, D), lambda i: (0, 0)),
        pl.BlockSpec((1, 8), lambda i: (0, 0)),
    ]
    out_specs = [pl.BlockSpec((_BLK, D), lambda i: (i, 0))]
    out_shape = [jax.ShapeDtypeStruct((np_, D), jnp.float32)]
    if emit_g:
        out_specs.append(pl.BlockSpec((_BLK, D), lambda i: (i, 0)))
        out_shape.append(jax.ShapeDtypeStruct((np_, D), jnp.float32))

    return pl.pallas_call(
        body,
        grid=(grid,),
        in_specs=in_specs,
        out_specs=out_specs,
        out_shape=out_shape,
    )(h, s2, s2, scal, wc, bc, lng, lnb, mix)


def _tc_pool_partial(part, batchp, np_):
    grid = np_ // _BLK

    def body(p_ref, b_ref, out_ref, pooled):
        i = pl.program_id(0)

        @pl.when(i == 0)
        def _():
            pooled[...] = jnp.zeros_like(pooled)

        col = lax.broadcasted_iota(jnp.int32, (_BLK, G_OUT), 1)
        m = (b_ref[...] == col).astype(jnp.float32)
        pooled[...] += lax.dot_general(
            m, p_ref[...], (((0,), (0,)), ((), ())),
            preferred_element_type=jnp.float32)

        @pl.when(i == grid - 1)
        def _():
            out_ref[...] = pooled[...]

    return pl.pallas_call(
        body,
        grid=(grid,),
        in_specs=[
            pl.BlockSpec((_BLK, D), lambda i: (i, 0)),
            pl.BlockSpec((_BLK, 1), lambda i: (i, 0)),
        ],
        out_specs=pl.BlockSpec((G_OUT, D), lambda i: (0, 0)),
        out_shape=jax.ShapeDtypeStruct((G_OUT, D), jnp.float32),
        scratch_shapes=[pltpu.VMEM((G_OUT, D), jnp.float32)],
    )(part, batchp)


def _tc_pool_post(h2, batchp, pool1, q1, qb1, q2, qb2, np_):
    grid = np_ // _BLK

    def body(h2_ref, b_ref, p1_ref, q1_ref, qb1_ref, q2_ref, qb2_ref,
             out_ref, pooled):
        i = pl.program_id(0)

        @pl.when(i == 0)
        def _():
            pooled[...] = p1_ref[...]

        col = lax.broadcasted_iota(jnp.int32, (_BLK, G_OUT), 1)
        m = (b_ref[...] == col).astype(jnp.float32)
        pooled[...] += lax.dot_general(
            m, h2_ref[...], (((0,), (0,)), ((), ())),
            preferred_element_type=jnp.float32)

        @pl.when(i == grid - 1)
        def _():
            t = jnp.dot(pooled[...], q1_ref[...],
                        preferred_element_type=jnp.float32) + qb1_ref[...]
            t = jnp.maximum(t, 0.0)
            out_ref[...] = jnp.dot(t, q2_ref[...],
                                   preferred_element_type=jnp.float32) \
                + qb2_ref[...]

    return pl.pallas_call(
        body,
        grid=(grid,),
        in_specs=[
            pl.BlockSpec((_BLK, D), lambda i: (i, 0)),
            pl.BlockSpec((_BLK, 1), lambda i: (i, 0)),
            pl.BlockSpec((G_OUT, D), lambda i: (0, 0)),
            pl.BlockSpec((D, D), lambda i: (0, 0)),
            pl.BlockSpec((1, D), lambda i: (0, 0)),
            pl.BlockSpec((D, D), lambda i: (0, 0)),
            pl.BlockSpec((1, D), lambda i: (0, 0)),
        ],
        out_specs=pl.BlockSpec((G_OUT, D), lambda i: (0, 0)),
        out_shape=jax.ShapeDtypeStruct((G_OUT, D), jnp.float32),
        scratch_shapes=[pltpu.VMEM((G_OUT, D), jnp.float32)],
    )(h2, batchp, pool1, q1, qb1.reshape(1, D), q2, qb2.reshape(1, D))


def _layer_weights(lp):
    ac = jax.nn.softmax(lp["alpha_conv"] / 1.0)
    an = jax.nn.softmax(lp["alpha_norm"] / 1.0)
    aa = jax.nn.softmax(lp["alpha_act"] / 1.0)
    wc = jnp.concatenate([ac[0] * lp["gcn"]["W"],
                          ac[1] * lp["sage"]["Ws"],
                          ac[1] * lp["sage"]["Wn"]], axis=0)
    bc = (ac[0] * lp["gcn"]["b"] + ac[1] * lp["sage"]["b"]).reshape(1, D)
    lng = lp["ln"]["g"].reshape(1, D)
    lnb = lp["ln"]["b"].reshape(1, D)
    mix = jnp.stack([an[0], an[1], aa[0], aa[1],
                     jnp.float32(0), jnp.float32(0),
                     jnp.float32(0), jnp.float32(0)]).reshape(1, 8)
    return wc, bc, lng, lnb, mix


def kernel(x, params, edge_index, batch):
    n, _ = x.shape
    e = edge_index.shape[1]
    np_ = _round_up(n + 1, _BLK)
    ep = _round_up(e, NW * SUPER * CHUNK)

    xp = jnp.pad(x, ((0, np_ - n), (0, 0)))
    pad_i = jnp.arange(ep - e, dtype=jnp.int32)
    pad_src = pad_i % n
    pad_dst = n + pad_i % (np_ - n)
    srcm = jnp.concatenate([edge_index[0], pad_src]).reshape(
        ep // CHUNK, CHUNK)
    dstm = jnp.concatenate([edge_index[1], pad_dst]).reshape(
        ep // CHUNK, CHUNK)
    batchp = jnp.pad(batch, (0, np_ - n),
                     constant_values=G_OUT).reshape(np_, 1)
    zrow = jnp.zeros((np_ // NS, D), jnp.float32)
    z1d = jnp.zeros((np_,), jnp.float32)

    degp = _sc_deg(dstm, z1d, np_, ep)
    pre = params["pre"]
    h, g1, scal = _tc_pre_mlp(xp, pre["W1"], pre["b1"], pre["W2"],
                              pre["b2"], degp.T, np_)

    s1 = _sc_scatter_two_tables(h, g1, srcm, dstm, zrow, np_, ep)
    wc1, bc1, lng1, lnb1, mix1 = _layer_weights(params["layers"][0])
    h1, g2 = _tc_layer(h, s1, scal, wc1, bc1, lng1, lnb1, mix1, np_,
                       emit_g=True, add_skip=False)

    s2 = _sc_scatter_two_tables(h1, g2, srcm, dstm, zrow, np_, ep)
    pool1 = _tc_pool_partial(h1, batchp, np_)
    wc2, bc2, lng2, lnb2, mix2 = _layer_weights(params["layers"][1])
    (h2,) = _tc_layer(h1, s2, scal, wc2, bc2, lng2, lnb2, mix2, np_,
                      emit_g=False, add_skip=False)

    post = params["post"]
    return _tc_pool_post(h2, batchp, pool1, post["W1"], post["b1"],
                         post["W2"], post["b2"], np_)
